# baseline (speedup 1.0000x reference)
import jax
import jax.numpy as jnp
import numpy as np
from jax.experimental import pallas as pl
from jax.experimental.pallas import tpu as pltpu

_EPS = 1e-5
_DIM = 4
_TRIL_R, _TRIL_C = np.tril_indices(_DIM)
_CORES = 2
_BB = 256
_G = 16


def _stats_kernel(x_ref, m2_ref, m1_ref):
    j = pl.program_id(1)
    first = j == 0
    for g in range(_G):
        xg = x_ref[:, g, :]
        m2 = jax.lax.dot_general(
            xg, xg, (((0,), (0,)), ((), ())),
            preferred_element_type=jnp.float32)
        m1 = jnp.sum(xg, axis=0, keepdims=True)

        @pl.when(first)
        def _():
            m2_ref[0, g] = m2
            m1_ref[0, g] = m1[0]

        @pl.when(jnp.logical_not(first))
        def _():
            m2_ref[0, g] = m2_ref[0, g] + m2
            m1_ref[0, g] = m1_ref[0, g] + m1[0]


def _make_solve_kernel(batch):
    inv_b = 1.0 / batch

    def _solve_kernel(p_ref, q_ref):
        def row(k):
            return p_ref[k:k + 1, :]

        m = [row(c) * inv_b for c in range(4)]
        cov = {}
        for k, (r, c) in enumerate(zip(_TRIL_R, _TRIL_C)):
            e = row(4 + k) * inv_b - m[r] * m[c]
            if r == c:
                e = e + _EPS
            cov[(r, c)] = e

        l00 = jnp.sqrt(cov[(0, 0)]); i0 = 1.0 / l00
        l10 = cov[(1, 0)] * i0
        l20 = cov[(2, 0)] * i0
        l30 = cov[(3, 0)] * i0
        l11 = jnp.sqrt(cov[(1, 1)] - l10 * l10); i1 = 1.0 / l11
        l21 = (cov[(2, 1)] - l20 * l10) * i1
        l31 = (cov[(3, 1)] - l30 * l10) * i1
        l22 = jnp.sqrt(cov[(2, 2)] - l20 * l20 - l21 * l21); i2 = 1.0 / l22
        l32 = (cov[(3, 2)] - l30 * l20 - l31 * l21) * i2
        l33 = jnp.sqrt(cov[(3, 3)] - l30 * l30 - l31 * l31 - l32 * l32)
        i3 = 1.0 / l33

        mm = {}
        mm[(0, 0)] = i0; mm[(1, 1)] = i1; mm[(2, 2)] = i2; mm[(3, 3)] = i3
        mm[(1, 0)] = -(l10 * mm[(0, 0)]) * i1
        mm[(2, 0)] = -(l20 * mm[(0, 0)] + l21 * mm[(1, 0)]) * i2
        mm[(2, 1)] = -(l21 * mm[(1, 1)]) * i2
        mm[(3, 0)] = -(l30 * mm[(0, 0)] + l31 * mm[(1, 0)] + l32 * mm[(2, 0)]) * i3
        mm[(3, 1)] = -(l31 * mm[(1, 1)] + l32 * mm[(2, 1)]) * i3
        mm[(3, 2)] = -(l32 * mm[(2, 2)]) * i3

        g = {}
        for k, (r, c) in enumerate(zip(_TRIL_R, _TRIL_C)):
            g[(r, c)] = row(14 + k)
            g[(c, r)] = g[(r, c)]

        a = {}
        for i in range(4):
            for jj in range(4):
                acc = None
                for k in range(jj, 4):
                    t = g[(i, k)] * mm[(k, jj)]
                    acc = t if acc is None else acc + t
                a[(i, jj)] = acc

        bias = []
        for i in range(4):
            s = row(24 + i)
            for jj in range(4):
                s = s - a[(i, jj)] * m[jj]
            bias.append(s)

        lanes = p_ref.shape[1]
        out_rows = [a[(i, jj)] for i in range(4) for jj in range(4)]
        out_rows += bias
        out_rows += [jnp.zeros((1, lanes), jnp.float32)] * 4
        q_ref[...] = jnp.concatenate(out_rows, axis=0)

    return _solve_kernel


def _apply_kernel(w_ref, b_ref, x_ref, o_ref):
    for g in range(_G):
        xg = x_ref[:, g, :]
        yg = jnp.dot(xg, w_ref[g],
                     preferred_element_type=jnp.float32)
        o_ref[:, g, :] = yg + b_ref[g:g + 1, :]


def kernel(x, gamma, beta):
    batch, nfeat, dim = x.shape
    f32 = jnp.float32
    xv = x.reshape(batch, _G, 128)
    n_inner = batch // (_CORES * _BB)

    cparams = pltpu.CompilerParams(
        dimension_semantics=("parallel", "arbitrary"))

    m2p, m1p = pl.pallas_call(
        _stats_kernel,
        grid=(_CORES, n_inner),
        in_specs=[pl.BlockSpec((_BB, _G, 128),
                               lambda c, j: (c * n_inner + j, 0, 0))],
        out_specs=[
            pl.BlockSpec((1, _G, 128, 128), lambda c, j: (c, 0, 0, 0)),
            pl.BlockSpec((1, _G, 128), lambda c, j: (c, 0, 0)),
        ],
        out_shape=[
            jax.ShapeDtypeStruct((_CORES, _G, 128, 128), f32),
            jax.ShapeDtypeStruct((_CORES, _G, 128), f32),
        ],
        compiler_params=cparams,
    )(xv)
    m2 = m2p[0] + m2p[1]
    m1 = m1p[0] + m1p[1]

    qidx = jnp.arange(32)
    d5 = m2.reshape(_G, 32, 4, 32, 4)
    diag = d5[:, qidx, :, qidx, :]
    covrows = jnp.stack(
        [diag[:, :, r, c].T.reshape(nfeat) for r, c in zip(_TRIL_R, _TRIL_C)],
        axis=0)
    s1 = m1.reshape(_G, 32, 4).transpose(2, 0, 1).reshape(4, nfeat)
    p_in = jnp.concatenate(
        [s1, covrows, gamma.T, beta.T, jnp.zeros((4, nfeat), f32)], axis=0)

    q = pl.pallas_call(
        _make_solve_kernel(batch),
        out_shape=jax.ShapeDtypeStruct((24, nfeat), f32),
    )(p_in)

    a_mat = q[:16].reshape(4, 4, _G, 32)
    blk = a_mat.transpose(2, 3, 1, 0)
    eye32 = jnp.eye(32, dtype=f32)
    w = (blk[:, :, :, None, :] *
         eye32[None, :, None, :, None]).reshape(_G, 128, 128)
    b_tile = q[16:20].reshape(4, _G, 32).transpose(1, 2, 0).reshape(_G, 128)

    out = pl.pallas_call(
        _apply_kernel,
        grid=(_CORES, n_inner),
        in_specs=[
            pl.BlockSpec((_G, 128, 128), lambda c, j: (0, 0, 0)),
            pl.BlockSpec((_G, 128), lambda c, j: (0, 0)),
            pl.BlockSpec((_BB, _G, 128),
                         lambda c, j: (c * n_inner + j, 0, 0)),
        ],
        out_specs=pl.BlockSpec((_BB, _G, 128),
                               lambda c, j: (c * n_inner + j, 0, 0)),
        out_shape=jax.ShapeDtypeStruct((batch, _G, 128), f32),
        compiler_params=cparams,
    )(w, b_tile, xv)
    return out.reshape(batch, nfeat, dim)


# speedup vs baseline: 4.5597x; 4.5597x over previous
import jax
import jax.numpy as jnp
import numpy as np
from jax.experimental import pallas as pl
from jax.experimental.pallas import tpu as pltpu

_EPS = 1e-5
_DIM = 4
_TRIL_R, _TRIL_C = np.tril_indices(_DIM)
_CORES = 2
_BB = 256


_CH = 16


def _stats_kernel(x_ref, m1_ref, m2_ref):
    j = pl.program_id(1)
    first = j == 0
    nf = x_ref.shape[2]
    zero = jnp.zeros((_CH, nf), jnp.float32)
    m1acc = [zero] * 4
    m2acc = [zero] * 10
    for c in range(_BB // _CH):
        xd = [x_ref[c * _CH:(c + 1) * _CH, d, :] for d in range(4)]
        for k, (r, cc) in enumerate(zip(_TRIL_R, _TRIL_C)):
            m2acc[k] = m2acc[k] + xd[r] * xd[cc]
        for d in range(4):
            m1acc[d] = m1acc[d] + xd[d]
    m1 = jnp.concatenate(
        [jnp.sum(a, axis=0, keepdims=True) for a in m1acc]
        + [jnp.zeros((4, nf), jnp.float32)], axis=0)
    m2 = jnp.concatenate(
        [jnp.sum(a, axis=0, keepdims=True) for a in m2acc]
        + [jnp.zeros((6, nf), jnp.float32)], axis=0)

    @pl.when(first)
    def _():
        m1_ref[0] = m1
        m2_ref[0] = m2

    @pl.when(jnp.logical_not(first))
    def _():
        m1_ref[0] = m1_ref[0] + m1
        m2_ref[0] = m2_ref[0] + m2


def _make_solve_kernel(batch):
    inv_b = 1.0 / batch

    def _solve_kernel(p_ref, q_ref):
        def row(k):
            return p_ref[k:k + 1, :]

        m = [row(c) * inv_b for c in range(4)]
        cov = {}
        for k, (r, c) in enumerate(zip(_TRIL_R, _TRIL_C)):
            e = row(4 + k) * inv_b - m[r] * m[c]
            if r == c:
                e = e + _EPS
            cov[(r, c)] = e

        l00 = jnp.sqrt(cov[(0, 0)]); i0 = 1.0 / l00
        l10 = cov[(1, 0)] * i0
        l20 = cov[(2, 0)] * i0
        l30 = cov[(3, 0)] * i0
        l11 = jnp.sqrt(cov[(1, 1)] - l10 * l10); i1 = 1.0 / l11
        l21 = (cov[(2, 1)] - l20 * l10) * i1
        l31 = (cov[(3, 1)] - l30 * l10) * i1
        l22 = jnp.sqrt(cov[(2, 2)] - l20 * l20 - l21 * l21); i2 = 1.0 / l22
        l32 = (cov[(3, 2)] - l30 * l20 - l31 * l21) * i2
        l33 = jnp.sqrt(cov[(3, 3)] - l30 * l30 - l31 * l31 - l32 * l32)
        i3 = 1.0 / l33

        mm = {}
        mm[(0, 0)] = i0; mm[(1, 1)] = i1; mm[(2, 2)] = i2; mm[(3, 3)] = i3
        mm[(1, 0)] = -(l10 * mm[(0, 0)]) * i1
        mm[(2, 0)] = -(l20 * mm[(0, 0)] + l21 * mm[(1, 0)]) * i2
        mm[(2, 1)] = -(l21 * mm[(1, 1)]) * i2
        mm[(3, 0)] = -(l30 * mm[(0, 0)] + l31 * mm[(1, 0)] + l32 * mm[(2, 0)]) * i3
        mm[(3, 1)] = -(l31 * mm[(1, 1)] + l32 * mm[(2, 1)]) * i3
        mm[(3, 2)] = -(l32 * mm[(2, 2)]) * i3

        g = {}
        for k, (r, c) in enumerate(zip(_TRIL_R, _TRIL_C)):
            g[(r, c)] = row(14 + k)
            g[(c, r)] = g[(r, c)]

        a = {}
        for i in range(4):
            for jj in range(4):
                acc = None
                for k in range(jj, 4):
                    t = g[(i, k)] * mm[(k, jj)]
                    acc = t if acc is None else acc + t
                a[(i, jj)] = acc

        bias = []
        for i in range(4):
            s = row(24 + i)
            for jj in range(4):
                s = s - a[(i, jj)] * m[jj]
            bias.append(s)

        lanes = p_ref.shape[1]
        out_rows = [a[(i, jj)] for i in range(4) for jj in range(4)]
        out_rows += bias
        out_rows += [jnp.zeros((1, lanes), jnp.float32)] * 4
        q_ref[...] = jnp.concatenate(out_rows, axis=0)

    return _solve_kernel


def _apply_kernel(cb_ref, x_ref, o_ref):
    xb = x_ref[...]
    acc = cb_ref[16:20, :][None] + cb_ref[0:4, :][None] * xb
    for s in range(1, 4):
        xs = pltpu.roll(xb, 4 - s, axis=1)
        acc = acc + cb_ref[4 * s:4 * s + 4, :][None] * xs
    o_ref[...] = acc


def kernel(x, gamma, beta):
    batch, nfeat, dim = x.shape
    f32 = jnp.float32
    xt = jnp.transpose(x, (0, 2, 1))
    n_inner = batch // (_CORES * _BB)

    cparams = pltpu.CompilerParams(
        dimension_semantics=("parallel", "arbitrary"))

    m1p, m2p = pl.pallas_call(
        _stats_kernel,
        grid=(_CORES, n_inner),
        in_specs=[pl.BlockSpec((_BB, 4, nfeat),
                               lambda c, j: (c * n_inner + j, 0, 0))],
        out_specs=[
            pl.BlockSpec((1, 8, nfeat), lambda c, j: (c, 0, 0)),
            pl.BlockSpec((1, 16, nfeat), lambda c, j: (c, 0, 0)),
        ],
        out_shape=[
            jax.ShapeDtypeStruct((_CORES, 8, nfeat), f32),
            jax.ShapeDtypeStruct((_CORES, 16, nfeat), f32),
        ],
        compiler_params=cparams,
    )(xt)
    m1 = m1p[0, :4] + m1p[1, :4]
    m2 = m2p[0] + m2p[1]

    p_in = jnp.concatenate(
        [m1, m2[:10], gamma.T, beta.T, jnp.zeros((4, nfeat), f32)], axis=0)

    q = pl.pallas_call(
        _make_solve_kernel(batch),
        out_shape=jax.ShapeDtypeStruct((24, nfeat), f32),
    )(p_in)

    a_mat = q[:16].reshape(4, 4, nfeat)
    cb_rows = [a_mat[i, (i + s) % 4] for s in range(4) for i in range(4)]
    cb = jnp.concatenate([jnp.stack(cb_rows, axis=0), q[16:20]], axis=0)

    out = pl.pallas_call(
        _apply_kernel,
        grid=(_CORES, n_inner),
        in_specs=[
            pl.BlockSpec((24, nfeat), lambda c, j: (0, 0)),
            pl.BlockSpec((_BB, 4, nfeat),
                         lambda c, j: (c * n_inner + j, 0, 0)),
        ],
        out_specs=pl.BlockSpec((_BB, 4, nfeat),
                               lambda c, j: (c * n_inner + j, 0, 0)),
        out_shape=jax.ShapeDtypeStruct((batch, 4, nfeat), f32),
        compiler_params=cparams,
    )(cb, xt)
    return jnp.transpose(out, (0, 2, 1))


# speedup vs baseline: 4.8640x; 1.0667x over previous
import jax
import jax.numpy as jnp
import numpy as np
from jax.experimental import pallas as pl
from jax.experimental.pallas import tpu as pltpu

_EPS = 1e-5
_DIM = 4
_TRIL_R, _TRIL_C = np.tril_indices(_DIM)
_CORES = 2
_BB = 512


_CH = 16


def _stats_kernel(x_ref, m1_ref, m2_ref):
    j = pl.program_id(1)
    first = j == 0
    nf = x_ref.shape[2]
    zero = jnp.zeros((_CH, nf), jnp.float32)
    m1acc = [zero] * 4
    m2acc = [zero] * 10
    for c in range(_BB // _CH):
        xd = [x_ref[c * _CH:(c + 1) * _CH, d, :] for d in range(4)]
        for k, (r, cc) in enumerate(zip(_TRIL_R, _TRIL_C)):
            m2acc[k] = m2acc[k] + xd[r] * xd[cc]
        for d in range(4):
            m1acc[d] = m1acc[d] + xd[d]
    m1 = jnp.concatenate(
        [jnp.sum(a, axis=0, keepdims=True) for a in m1acc]
        + [jnp.zeros((4, nf), jnp.float32)], axis=0)
    m2 = jnp.concatenate(
        [jnp.sum(a, axis=0, keepdims=True) for a in m2acc]
        + [jnp.zeros((6, nf), jnp.float32)], axis=0)

    @pl.when(first)
    def _():
        m1_ref[0] = m1
        m2_ref[0] = m2

    @pl.when(jnp.logical_not(first))
    def _():
        m1_ref[0] = m1_ref[0] + m1
        m2_ref[0] = m2_ref[0] + m2


def _make_solve_kernel(batch):
    inv_b = 1.0 / batch

    def _solve_kernel(p_ref, q_ref):
        def row(k):
            return p_ref[k:k + 1, :]

        m = [row(c) * inv_b for c in range(4)]
        cov = {}
        for k, (r, c) in enumerate(zip(_TRIL_R, _TRIL_C)):
            e = row(4 + k) * inv_b - m[r] * m[c]
            if r == c:
                e = e + _EPS
            cov[(r, c)] = e

        l00 = jnp.sqrt(cov[(0, 0)]); i0 = 1.0 / l00
        l10 = cov[(1, 0)] * i0
        l20 = cov[(2, 0)] * i0
        l30 = cov[(3, 0)] * i0
        l11 = jnp.sqrt(cov[(1, 1)] - l10 * l10); i1 = 1.0 / l11
        l21 = (cov[(2, 1)] - l20 * l10) * i1
        l31 = (cov[(3, 1)] - l30 * l10) * i1
        l22 = jnp.sqrt(cov[(2, 2)] - l20 * l20 - l21 * l21); i2 = 1.0 / l22
        l32 = (cov[(3, 2)] - l30 * l20 - l31 * l21) * i2
        l33 = jnp.sqrt(cov[(3, 3)] - l30 * l30 - l31 * l31 - l32 * l32)
        i3 = 1.0 / l33

        mm = {}
        mm[(0, 0)] = i0; mm[(1, 1)] = i1; mm[(2, 2)] = i2; mm[(3, 3)] = i3
        mm[(1, 0)] = -(l10 * mm[(0, 0)]) * i1
        mm[(2, 0)] = -(l20 * mm[(0, 0)] + l21 * mm[(1, 0)]) * i2
        mm[(2, 1)] = -(l21 * mm[(1, 1)]) * i2
        mm[(3, 0)] = -(l30 * mm[(0, 0)] + l31 * mm[(1, 0)] + l32 * mm[(2, 0)]) * i3
        mm[(3, 1)] = -(l31 * mm[(1, 1)] + l32 * mm[(2, 1)]) * i3
        mm[(3, 2)] = -(l32 * mm[(2, 2)]) * i3

        g = {}
        for k, (r, c) in enumerate(zip(_TRIL_R, _TRIL_C)):
            g[(r, c)] = row(14 + k)
            g[(c, r)] = g[(r, c)]

        a = {}
        for i in range(4):
            for jj in range(4):
                acc = None
                for k in range(jj, 4):
                    t = g[(i, k)] * mm[(k, jj)]
                    acc = t if acc is None else acc + t
                a[(i, jj)] = acc

        bias = []
        for i in range(4):
            s = row(24 + i)
            for jj in range(4):
                s = s - a[(i, jj)] * m[jj]
            bias.append(s)

        lanes = p_ref.shape[1]
        out_rows = [a[(i, jj)] for i in range(4) for jj in range(4)]
        out_rows += bias
        out_rows += [jnp.zeros((1, lanes), jnp.float32)] * 4
        q_ref[...] = jnp.concatenate(out_rows, axis=0)

    return _solve_kernel


def _apply_kernel(cb_ref, x_ref, o_ref):
    xb = x_ref[...]
    acc = cb_ref[16:20, :][None] + cb_ref[0:4, :][None] * xb
    for s in range(1, 4):
        xs = pltpu.roll(xb, 4 - s, axis=1)
        acc = acc + cb_ref[4 * s:4 * s + 4, :][None] * xs
    o_ref[...] = acc


def kernel(x, gamma, beta):
    batch, nfeat, dim = x.shape
    f32 = jnp.float32
    xt = jnp.transpose(x, (0, 2, 1))
    n_inner = batch // (_CORES * _BB)

    cparams = pltpu.CompilerParams(
        dimension_semantics=("parallel", "arbitrary"))

    m1p, m2p = pl.pallas_call(
        _stats_kernel,
        grid=(_CORES, n_inner),
        in_specs=[pl.BlockSpec((_BB, 4, nfeat),
                               lambda c, j: (c * n_inner + j, 0, 0))],
        out_specs=[
            pl.BlockSpec((1, 8, nfeat), lambda c, j: (c, 0, 0)),
            pl.BlockSpec((1, 16, nfeat), lambda c, j: (c, 0, 0)),
        ],
        out_shape=[
            jax.ShapeDtypeStruct((_CORES, 8, nfeat), f32),
            jax.ShapeDtypeStruct((_CORES, 16, nfeat), f32),
        ],
        compiler_params=cparams,
    )(xt)
    m1 = m1p[0, :4] + m1p[1, :4]
    m2 = m2p[0] + m2p[1]

    p_in = jnp.concatenate(
        [m1, m2[:10], gamma.T, beta.T, jnp.zeros((4, nfeat), f32)], axis=0)

    q = pl.pallas_call(
        _make_solve_kernel(batch),
        out_shape=jax.ShapeDtypeStruct((24, nfeat), f32),
    )(p_in)

    a_mat = q[:16].reshape(4, 4, nfeat)
    cb_rows = [a_mat[i, (i + s) % 4] for s in range(4) for i in range(4)]
    cb = jnp.concatenate([jnp.stack(cb_rows, axis=0), q[16:20]], axis=0)

    out = pl.pallas_call(
        _apply_kernel,
        grid=(_CORES, n_inner),
        in_specs=[
            pl.BlockSpec((24, nfeat), lambda c, j: (0, 0)),
            pl.BlockSpec((_BB, 4, nfeat),
                         lambda c, j: (c * n_inner + j, 0, 0)),
        ],
        out_specs=pl.BlockSpec((_BB, 4, nfeat),
                               lambda c, j: (c * n_inner + j, 0, 0)),
        out_shape=jax.ShapeDtypeStruct((batch, 4, nfeat), f32),
        compiler_params=cparams,
    )(cb, xt)
    return jnp.transpose(out, (0, 2, 1))


# speedup vs baseline: 4.8759x; 1.0024x over previous
import jax
import jax.numpy as jnp
import numpy as np
from jax.experimental import pallas as pl
from jax.experimental.pallas import tpu as pltpu

_EPS = 1e-5
_DIM = 4
_TRIL_R, _TRIL_C = np.tril_indices(_DIM)
_CORES = 2
_BB = 1024


_CH = 16


def _stats_kernel(x_ref, m1_ref, m2_ref):
    j = pl.program_id(1)
    first = j == 0
    nf = x_ref.shape[2]
    zero = jnp.zeros((_CH, nf), jnp.float32)
    m1acc = [zero] * 4
    m2acc = [zero] * 10
    for c in range(_BB // _CH):
        xd = [x_ref[c * _CH:(c + 1) * _CH, d, :] for d in range(4)]
        for k, (r, cc) in enumerate(zip(_TRIL_R, _TRIL_C)):
            m2acc[k] = m2acc[k] + xd[r] * xd[cc]
        for d in range(4):
            m1acc[d] = m1acc[d] + xd[d]
    m1 = jnp.concatenate(
        [jnp.sum(a, axis=0, keepdims=True) for a in m1acc]
        + [jnp.zeros((4, nf), jnp.float32)], axis=0)
    m2 = jnp.concatenate(
        [jnp.sum(a, axis=0, keepdims=True) for a in m2acc]
        + [jnp.zeros((6, nf), jnp.float32)], axis=0)

    @pl.when(first)
    def _():
        m1_ref[0] = m1
        m2_ref[0] = m2

    @pl.when(jnp.logical_not(first))
    def _():
        m1_ref[0] = m1_ref[0] + m1
        m2_ref[0] = m2_ref[0] + m2


def _make_solve_kernel(batch):
    inv_b = 1.0 / batch

    def _solve_kernel(p_ref, q_ref):
        def row(k):
            return p_ref[k:k + 1, :]

        m = [row(c) * inv_b for c in range(4)]
        cov = {}
        for k, (r, c) in enumerate(zip(_TRIL_R, _TRIL_C)):
            e = row(4 + k) * inv_b - m[r] * m[c]
            if r == c:
                e = e + _EPS
            cov[(r, c)] = e

        l00 = jnp.sqrt(cov[(0, 0)]); i0 = 1.0 / l00
        l10 = cov[(1, 0)] * i0
        l20 = cov[(2, 0)] * i0
        l30 = cov[(3, 0)] * i0
        l11 = jnp.sqrt(cov[(1, 1)] - l10 * l10); i1 = 1.0 / l11
        l21 = (cov[(2, 1)] - l20 * l10) * i1
        l31 = (cov[(3, 1)] - l30 * l10) * i1
        l22 = jnp.sqrt(cov[(2, 2)] - l20 * l20 - l21 * l21); i2 = 1.0 / l22
        l32 = (cov[(3, 2)] - l30 * l20 - l31 * l21) * i2
        l33 = jnp.sqrt(cov[(3, 3)] - l30 * l30 - l31 * l31 - l32 * l32)
        i3 = 1.0 / l33

        mm = {}
        mm[(0, 0)] = i0; mm[(1, 1)] = i1; mm[(2, 2)] = i2; mm[(3, 3)] = i3
        mm[(1, 0)] = -(l10 * mm[(0, 0)]) * i1
        mm[(2, 0)] = -(l20 * mm[(0, 0)] + l21 * mm[(1, 0)]) * i2
        mm[(2, 1)] = -(l21 * mm[(1, 1)]) * i2
        mm[(3, 0)] = -(l30 * mm[(0, 0)] + l31 * mm[(1, 0)] + l32 * mm[(2, 0)]) * i3
        mm[(3, 1)] = -(l31 * mm[(1, 1)] + l32 * mm[(2, 1)]) * i3
        mm[(3, 2)] = -(l32 * mm[(2, 2)]) * i3

        g = {}
        for k, (r, c) in enumerate(zip(_TRIL_R, _TRIL_C)):
            g[(r, c)] = row(14 + k)
            g[(c, r)] = g[(r, c)]

        a = {}
        for i in range(4):
            for jj in range(4):
                acc = None
                for k in range(jj, 4):
                    t = g[(i, k)] * mm[(k, jj)]
                    acc = t if acc is None else acc + t
                a[(i, jj)] = acc

        bias = []
        for i in range(4):
            s = row(24 + i)
            for jj in range(4):
                s = s - a[(i, jj)] * m[jj]
            bias.append(s)

        lanes = p_ref.shape[1]
        out_rows = [a[(i, jj)] for i in range(4) for jj in range(4)]
        out_rows += bias
        out_rows += [jnp.zeros((1, lanes), jnp.float32)] * 4
        q_ref[...] = jnp.concatenate(out_rows, axis=0)

    return _solve_kernel


def _apply_kernel(cb_ref, x_ref, o_ref):
    xb = x_ref[...]
    acc = cb_ref[16:20, :][None] + cb_ref[0:4, :][None] * xb
    for s in range(1, 4):
        xs = pltpu.roll(xb, 4 - s, axis=1)
        acc = acc + cb_ref[4 * s:4 * s + 4, :][None] * xs
    o_ref[...] = acc


def kernel(x, gamma, beta):
    batch, nfeat, dim = x.shape
    f32 = jnp.float32
    xt = jnp.transpose(x, (0, 2, 1))
    n_inner = batch // (_CORES * _BB)

    cparams = pltpu.CompilerParams(
        dimension_semantics=("parallel", "arbitrary"),
        vmem_limit_bytes=100 * 1024 * 1024)

    m1p, m2p = pl.pallas_call(
        _stats_kernel,
        grid=(_CORES, n_inner),
        in_specs=[pl.BlockSpec((_BB, 4, nfeat),
                               lambda c, j: (c * n_inner + j, 0, 0))],
        out_specs=[
            pl.BlockSpec((1, 8, nfeat), lambda c, j: (c, 0, 0)),
            pl.BlockSpec((1, 16, nfeat), lambda c, j: (c, 0, 0)),
        ],
        out_shape=[
            jax.ShapeDtypeStruct((_CORES, 8, nfeat), f32),
            jax.ShapeDtypeStruct((_CORES, 16, nfeat), f32),
        ],
        compiler_params=cparams,
    )(xt)
    m1 = m1p[0, :4] + m1p[1, :4]
    m2 = m2p[0] + m2p[1]

    p_in = jnp.concatenate(
        [m1, m2[:10], gamma.T, beta.T, jnp.zeros((4, nfeat), f32)], axis=0)

    q = pl.pallas_call(
        _make_solve_kernel(batch),
        out_shape=jax.ShapeDtypeStruct((24, nfeat), f32),
    )(p_in)

    a_mat = q[:16].reshape(4, 4, nfeat)
    cb_rows = [a_mat[i, (i + s) % 4] for s in range(4) for i in range(4)]
    cb = jnp.concatenate([jnp.stack(cb_rows, axis=0), q[16:20]], axis=0)

    out = pl.pallas_call(
        _apply_kernel,
        grid=(_CORES, n_inner),
        in_specs=[
            pl.BlockSpec((24, nfeat), lambda c, j: (0, 0)),
            pl.BlockSpec((_BB, 4, nfeat),
                         lambda c, j: (c * n_inner + j, 0, 0)),
        ],
        out_specs=pl.BlockSpec((_BB, 4, nfeat),
                               lambda c, j: (c * n_inner + j, 0, 0)),
        out_shape=jax.ShapeDtypeStruct((batch, 4, nfeat), f32),
        compiler_params=cparams,
    )(cb, xt)
    return jnp.transpose(out, (0, 2, 1))


# speedup vs baseline: 5.1118x; 1.0484x over previous
import jax
import jax.numpy as jnp
import numpy as np
from jax.experimental import pallas as pl
from jax.experimental.pallas import tpu as pltpu

_EPS = 1e-5
_DIM = 4
_TRIL_R, _TRIL_C = np.tril_indices(_DIM)
_CORES = 2
_BB = 256


_CH = 16


def _stats_kernel(x_ref, m1_ref, m2_ref):
    j = pl.program_id(1)
    first = j == 0
    nf = x_ref.shape[2]
    zero = jnp.zeros((_CH, nf), jnp.float32)
    m1acc = [zero] * 4
    m2acc = [zero] * 10
    for c in range(_BB // _CH):
        xd = [x_ref[c * _CH:(c + 1) * _CH, d, :] for d in range(4)]
        for k, (r, cc) in enumerate(zip(_TRIL_R, _TRIL_C)):
            m2acc[k] = m2acc[k] + xd[r] * xd[cc]
        for d in range(4):
            m1acc[d] = m1acc[d] + xd[d]
    m1 = jnp.concatenate(
        [jnp.sum(a, axis=0, keepdims=True) for a in m1acc]
        + [jnp.zeros((4, nf), jnp.float32)], axis=0)
    m2 = jnp.concatenate(
        [jnp.sum(a, axis=0, keepdims=True) for a in m2acc]
        + [jnp.zeros((6, nf), jnp.float32)], axis=0)

    @pl.when(first)
    def _():
        m1_ref[0] = m1
        m2_ref[0] = m2

    @pl.when(jnp.logical_not(first))
    def _():
        m1_ref[0] = m1_ref[0] + m1
        m2_ref[0] = m2_ref[0] + m2


def _make_solve_kernel(batch):
    inv_b = 1.0 / batch

    def _solve_kernel(p_ref, q_ref):
        def row(k):
            return p_ref[k:k + 1, :]

        m = [row(c) * inv_b for c in range(4)]
        cov = {}
        for k, (r, c) in enumerate(zip(_TRIL_R, _TRIL_C)):
            e = row(4 + k) * inv_b - m[r] * m[c]
            if r == c:
                e = e + _EPS
            cov[(r, c)] = e

        l00 = jnp.sqrt(cov[(0, 0)]); i0 = 1.0 / l00
        l10 = cov[(1, 0)] * i0
        l20 = cov[(2, 0)] * i0
        l30 = cov[(3, 0)] * i0
        l11 = jnp.sqrt(cov[(1, 1)] - l10 * l10); i1 = 1.0 / l11
        l21 = (cov[(2, 1)] - l20 * l10) * i1
        l31 = (cov[(3, 1)] - l30 * l10) * i1
        l22 = jnp.sqrt(cov[(2, 2)] - l20 * l20 - l21 * l21); i2 = 1.0 / l22
        l32 = (cov[(3, 2)] - l30 * l20 - l31 * l21) * i2
        l33 = jnp.sqrt(cov[(3, 3)] - l30 * l30 - l31 * l31 - l32 * l32)
        i3 = 1.0 / l33

        mm = {}
        mm[(0, 0)] = i0; mm[(1, 1)] = i1; mm[(2, 2)] = i2; mm[(3, 3)] = i3
        mm[(1, 0)] = -(l10 * mm[(0, 0)]) * i1
        mm[(2, 0)] = -(l20 * mm[(0, 0)] + l21 * mm[(1, 0)]) * i2
        mm[(2, 1)] = -(l21 * mm[(1, 1)]) * i2
        mm[(3, 0)] = -(l30 * mm[(0, 0)] + l31 * mm[(1, 0)] + l32 * mm[(2, 0)]) * i3
        mm[(3, 1)] = -(l31 * mm[(1, 1)] + l32 * mm[(2, 1)]) * i3
        mm[(3, 2)] = -(l32 * mm[(2, 2)]) * i3

        g = {}
        for k, (r, c) in enumerate(zip(_TRIL_R, _TRIL_C)):
            g[(r, c)] = row(14 + k)
            g[(c, r)] = g[(r, c)]

        a = {}
        for i in range(4):
            for jj in range(4):
                acc = None
                for k in range(jj, 4):
                    t = g[(i, k)] * mm[(k, jj)]
                    acc = t if acc is None else acc + t
                a[(i, jj)] = acc

        bias = []
        for i in range(4):
            s = row(24 + i)
            for jj in range(4):
                s = s - a[(i, jj)] * m[jj]
            bias.append(s)

        lanes = p_ref.shape[1]
        out_rows = [a[(i, jj)] for i in range(4) for jj in range(4)]
        out_rows += bias
        out_rows += [jnp.zeros((1, lanes), jnp.float32)] * 4
        q_ref[...] = jnp.concatenate(out_rows, axis=0)

    return _solve_kernel


def _apply_kernel(cb_ref, x_ref, o_ref):
    a = [[cb_ref[4 * i + jj:4 * i + jj + 1, :] for jj in range(4)]
         for i in range(4)]
    bias = [cb_ref[16 + i:17 + i, :] for i in range(4)]
    for c in range(_BB // _CH):
        sl = slice(c * _CH, (c + 1) * _CH)
        xd = [x_ref[sl, d, :] for d in range(4)]
        for i in range(4):
            y = (bias[i] + a[i][0] * xd[0] + a[i][1] * xd[1]
                 + a[i][2] * xd[2] + a[i][3] * xd[3])
            o_ref[sl, i, :] = y


def kernel(x, gamma, beta):
    batch, nfeat, dim = x.shape
    f32 = jnp.float32
    xt = jnp.transpose(x, (0, 2, 1))
    n_inner = batch // (_CORES * _BB)

    cparams = pltpu.CompilerParams(
        dimension_semantics=("parallel", "arbitrary"),
        vmem_limit_bytes=100 * 1024 * 1024)

    m1p, m2p = pl.pallas_call(
        _stats_kernel,
        grid=(_CORES, n_inner),
        in_specs=[pl.BlockSpec((_BB, 4, nfeat),
                               lambda c, j: (c * n_inner + j, 0, 0))],
        out_specs=[
            pl.BlockSpec((1, 8, nfeat), lambda c, j: (c, 0, 0)),
            pl.BlockSpec((1, 16, nfeat), lambda c, j: (c, 0, 0)),
        ],
        out_shape=[
            jax.ShapeDtypeStruct((_CORES, 8, nfeat), f32),
            jax.ShapeDtypeStruct((_CORES, 16, nfeat), f32),
        ],
        compiler_params=cparams,
    )(xt)
    m1 = m1p[0, :4] + m1p[1, :4]
    m2 = m2p[0] + m2p[1]

    p_in = jnp.concatenate(
        [m1, m2[:10], gamma.T, beta.T, jnp.zeros((4, nfeat), f32)], axis=0)

    q = pl.pallas_call(
        _make_solve_kernel(batch),
        out_shape=jax.ShapeDtypeStruct((24, nfeat), f32),
    )(p_in)

    cb = q[:20]

    out = pl.pallas_call(
        _apply_kernel,
        grid=(_CORES, n_inner),
        in_specs=[
            pl.BlockSpec((24, nfeat), lambda c, j: (0, 0)),
            pl.BlockSpec((_BB, 4, nfeat),
                         lambda c, j: (c * n_inner + j, 0, 0)),
        ],
        out_specs=pl.BlockSpec((_BB, 4, nfeat),
                               lambda c, j: (c * n_inner + j, 0, 0)),
        out_shape=jax.ShapeDtypeStruct((batch, 4, nfeat), f32),
        compiler_params=cparams,
    )(cb, xt)
    return jnp.transpose(out, (0, 2, 1))


# speedup vs baseline: 6.0651x; 1.1865x over previous
import jax
import jax.numpy as jnp
import numpy as np
from jax.experimental import pallas as pl
from jax.experimental.pallas import tpu as pltpu

_EPS = 1e-5
_DIM = 4
_TRIL_R, _TRIL_C = np.tril_indices(_DIM)
_CORES = 2
_BB = 1024
_BBS = 2048


_CH = 16


def _stats_kernel(x_ref, m1_ref, m2_ref):
    j = pl.program_id(1)
    first = j == 0
    nf = x_ref.shape[2]
    zero = jnp.zeros((_CH, nf), jnp.float32)
    m1acc = [zero] * 4
    m2acc = [zero] * 10
    for c in range(_BBS // _CH):
        xd = [x_ref[c * _CH:(c + 1) * _CH, d, :] for d in range(4)]
        for k, (r, cc) in enumerate(zip(_TRIL_R, _TRIL_C)):
            m2acc[k] = m2acc[k] + xd[r] * xd[cc]
        for d in range(4):
            m1acc[d] = m1acc[d] + xd[d]
    m1 = jnp.concatenate(
        [jnp.sum(a, axis=0, keepdims=True) for a in m1acc]
        + [jnp.zeros((4, nf), jnp.float32)], axis=0)
    m2 = jnp.concatenate(
        [jnp.sum(a, axis=0, keepdims=True) for a in m2acc]
        + [jnp.zeros((6, nf), jnp.float32)], axis=0)

    @pl.when(first)
    def _():
        m1_ref[0] = m1
        m2_ref[0] = m2

    @pl.when(jnp.logical_not(first))
    def _():
        m1_ref[0] = m1_ref[0] + m1
        m2_ref[0] = m2_ref[0] + m2


def _make_solve_kernel(batch):
    inv_b = 1.0 / batch

    def _solve_kernel(p_ref, q_ref):
        def row(k):
            return p_ref[k:k + 1, :]

        m = [row(c) * inv_b for c in range(4)]
        cov = {}
        for k, (r, c) in enumerate(zip(_TRIL_R, _TRIL_C)):
            e = row(4 + k) * inv_b - m[r] * m[c]
            if r == c:
                e = e + _EPS
            cov[(r, c)] = e

        l00 = jnp.sqrt(cov[(0, 0)]); i0 = 1.0 / l00
        l10 = cov[(1, 0)] * i0
        l20 = cov[(2, 0)] * i0
        l30 = cov[(3, 0)] * i0
        l11 = jnp.sqrt(cov[(1, 1)] - l10 * l10); i1 = 1.0 / l11
        l21 = (cov[(2, 1)] - l20 * l10) * i1
        l31 = (cov[(3, 1)] - l30 * l10) * i1
        l22 = jnp.sqrt(cov[(2, 2)] - l20 * l20 - l21 * l21); i2 = 1.0 / l22
        l32 = (cov[(3, 2)] - l30 * l20 - l31 * l21) * i2
        l33 = jnp.sqrt(cov[(3, 3)] - l30 * l30 - l31 * l31 - l32 * l32)
        i3 = 1.0 / l33

        mm = {}
        mm[(0, 0)] = i0; mm[(1, 1)] = i1; mm[(2, 2)] = i2; mm[(3, 3)] = i3
        mm[(1, 0)] = -(l10 * mm[(0, 0)]) * i1
        mm[(2, 0)] = -(l20 * mm[(0, 0)] + l21 * mm[(1, 0)]) * i2
        mm[(2, 1)] = -(l21 * mm[(1, 1)]) * i2
        mm[(3, 0)] = -(l30 * mm[(0, 0)] + l31 * mm[(1, 0)] + l32 * mm[(2, 0)]) * i3
        mm[(3, 1)] = -(l31 * mm[(1, 1)] + l32 * mm[(2, 1)]) * i3
        mm[(3, 2)] = -(l32 * mm[(2, 2)]) * i3

        g = {}
        for k, (r, c) in enumerate(zip(_TRIL_R, _TRIL_C)):
            g[(r, c)] = row(14 + k)
            g[(c, r)] = g[(r, c)]

        a = {}
        for i in range(4):
            for jj in range(4):
                acc = None
                for k in range(jj, 4):
                    t = g[(i, k)] * mm[(k, jj)]
                    acc = t if acc is None else acc + t
                a[(i, jj)] = acc

        bias = []
        for i in range(4):
            s = row(24 + i)
            for jj in range(4):
                s = s - a[(i, jj)] * m[jj]
            bias.append(s)

        lanes = p_ref.shape[1]
        out_rows = [a[(i, jj)] for i in range(4) for jj in range(4)]
        out_rows += bias
        out_rows += [jnp.zeros((1, lanes), jnp.float32)] * 4
        q_ref[...] = jnp.concatenate(out_rows, axis=0)

    return _solve_kernel


def _apply_kernel(cb_ref, x_ref, o_ref):
    a = [[cb_ref[4 * i + jj:4 * i + jj + 1, :] for jj in range(4)]
         for i in range(4)]
    bias = [cb_ref[16 + i:17 + i, :] for i in range(4)]
    for c in range(_BB // _CH):
        sl = slice(c * _CH, (c + 1) * _CH)
        xd = [x_ref[sl, d, :] for d in range(4)]
        for i in range(4):
            y = (bias[i] + a[i][0] * xd[0] + a[i][1] * xd[1]
                 + a[i][2] * xd[2] + a[i][3] * xd[3])
            o_ref[sl, i, :] = y


def kernel(x, gamma, beta):
    batch, nfeat, dim = x.shape
    f32 = jnp.float32
    xt = jnp.transpose(x, (0, 2, 1))
    n_inner = batch // (_CORES * _BB)
    n_inner_s = batch // (_CORES * _BBS)

    cparams = pltpu.CompilerParams(
        dimension_semantics=("parallel", "arbitrary"),
        vmem_limit_bytes=100 * 1024 * 1024)

    m1p, m2p = pl.pallas_call(
        _stats_kernel,
        grid=(_CORES, n_inner_s),
        in_specs=[pl.BlockSpec((_BBS, 4, nfeat),
                               lambda c, j: (c * n_inner_s + j, 0, 0))],
        out_specs=[
            pl.BlockSpec((1, 8, nfeat), lambda c, j: (c, 0, 0)),
            pl.BlockSpec((1, 16, nfeat), lambda c, j: (c, 0, 0)),
        ],
        out_shape=[
            jax.ShapeDtypeStruct((_CORES, 8, nfeat), f32),
            jax.ShapeDtypeStruct((_CORES, 16, nfeat), f32),
        ],
        compiler_params=cparams,
    )(xt)
    m1 = m1p[0, :4] + m1p[1, :4]
    m2 = m2p[0] + m2p[1]

    p_in = jnp.concatenate(
        [m1, m2[:10], gamma.T, beta.T, jnp.zeros((4, nfeat), f32)], axis=0)

    q = pl.pallas_call(
        _make_solve_kernel(batch),
        out_shape=jax.ShapeDtypeStruct((24, nfeat), f32),
    )(p_in)

    cb = q[:20]

    out = pl.pallas_call(
        _apply_kernel,
        grid=(_CORES, n_inner),
        in_specs=[
            pl.BlockSpec((24, nfeat), lambda c, j: (0, 0)),
            pl.BlockSpec((_BB, 4, nfeat),
                         lambda c, j: (c * n_inner + j, 0, 0)),
        ],
        out_specs=pl.BlockSpec((_BB, 4, nfeat),
                               lambda c, j: (c * n_inner + j, 0, 0)),
        out_shape=jax.ShapeDtypeStruct((batch, 4, nfeat), f32),
        compiler_params=cparams,
    )(cb, xt)
    return jnp.transpose(out, (0, 2, 1))


# speedup vs baseline: 6.0760x; 1.0018x over previous
import jax
import jax.numpy as jnp
import numpy as np
from jax.experimental import pallas as pl
from jax.experimental.pallas import tpu as pltpu

_EPS = 1e-5
_DIM = 4
_TRIL_R, _TRIL_C = np.tril_indices(_DIM)
_CORES = 2
_BB = 1024


_CH = 16


def _stats_kernel(x_ref, m1_ref, m2_ref):
    j = pl.program_id(1)
    first = j == 0
    nf = x_ref.shape[2]
    zero = jnp.zeros((_CH, nf), jnp.float32)
    m1acc = [zero] * 4
    m2acc = [zero] * 10
    for c in range(_BB // _CH):
        xd = [x_ref[c * _CH:(c + 1) * _CH, d, :] for d in range(4)]
        for k, (r, cc) in enumerate(zip(_TRIL_R, _TRIL_C)):
            m2acc[k] = m2acc[k] + xd[r] * xd[cc]
        for d in range(4):
            m1acc[d] = m1acc[d] + xd[d]
    m1 = jnp.concatenate(
        [jnp.sum(a, axis=0, keepdims=True) for a in m1acc]
        + [jnp.zeros((4, nf), jnp.float32)], axis=0)
    m2 = jnp.concatenate(
        [jnp.sum(a, axis=0, keepdims=True) for a in m2acc]
        + [jnp.zeros((6, nf), jnp.float32)], axis=0)

    @pl.when(first)
    def _():
        m1_ref[0] = m1
        m2_ref[0] = m2

    @pl.when(jnp.logical_not(first))
    def _():
        m1_ref[0] = m1_ref[0] + m1
        m2_ref[0] = m2_ref[0] + m2


def _make_solve_kernel(batch):
    inv_b = 1.0 / batch

    def _solve_kernel(p_ref, q_ref):
        def row(k):
            return p_ref[k:k + 1, :]

        m = [row(c) * inv_b for c in range(4)]
        cov = {}
        for k, (r, c) in enumerate(zip(_TRIL_R, _TRIL_C)):
            e = row(4 + k) * inv_b - m[r] * m[c]
            if r == c:
                e = e + _EPS
            cov[(r, c)] = e

        l00 = jnp.sqrt(cov[(0, 0)]); i0 = 1.0 / l00
        l10 = cov[(1, 0)] * i0
        l20 = cov[(2, 0)] * i0
        l30 = cov[(3, 0)] * i0
        l11 = jnp.sqrt(cov[(1, 1)] - l10 * l10); i1 = 1.0 / l11
        l21 = (cov[(2, 1)] - l20 * l10) * i1
        l31 = (cov[(3, 1)] - l30 * l10) * i1
        l22 = jnp.sqrt(cov[(2, 2)] - l20 * l20 - l21 * l21); i2 = 1.0 / l22
        l32 = (cov[(3, 2)] - l30 * l20 - l31 * l21) * i2
        l33 = jnp.sqrt(cov[(3, 3)] - l30 * l30 - l31 * l31 - l32 * l32)
        i3 = 1.0 / l33

        mm = {}
        mm[(0, 0)] = i0; mm[(1, 1)] = i1; mm[(2, 2)] = i2; mm[(3, 3)] = i3
        mm[(1, 0)] = -(l10 * mm[(0, 0)]) * i1
        mm[(2, 0)] = -(l20 * mm[(0, 0)] + l21 * mm[(1, 0)]) * i2
        mm[(2, 1)] = -(l21 * mm[(1, 1)]) * i2
        mm[(3, 0)] = -(l30 * mm[(0, 0)] + l31 * mm[(1, 0)] + l32 * mm[(2, 0)]) * i3
        mm[(3, 1)] = -(l31 * mm[(1, 1)] + l32 * mm[(2, 1)]) * i3
        mm[(3, 2)] = -(l32 * mm[(2, 2)]) * i3

        g = {}
        for k, (r, c) in enumerate(zip(_TRIL_R, _TRIL_C)):
            g[(r, c)] = row(14 + k)
            g[(c, r)] = g[(r, c)]

        a = {}
        for i in range(4):
            for jj in range(4):
                acc = None
                for k in range(jj, 4):
                    t = g[(i, k)] * mm[(k, jj)]
                    acc = t if acc is None else acc + t
                a[(i, jj)] = acc

        bias = []
        for i in range(4):
            s = row(24 + i)
            for jj in range(4):
                s = s - a[(i, jj)] * m[jj]
            bias.append(s)

        lanes = p_ref.shape[1]
        out_rows = [a[(i, jj)] for i in range(4) for jj in range(4)]
        out_rows += bias
        out_rows += [jnp.zeros((1, lanes), jnp.float32)] * 4
        q_ref[...] = jnp.concatenate(out_rows, axis=0)

    return _solve_kernel


def _apply_kernel(cb_ref, x_ref, o_ref):
    a = [[cb_ref[4 * i + jj:4 * i + jj + 1, :] for jj in range(4)]
         for i in range(4)]
    bias = [cb_ref[16 + i:17 + i, :] for i in range(4)]
    for c in range(_BB // _CH):
        sl = slice(c * _CH, (c + 1) * _CH)
        xd = [x_ref[sl, d, :] for d in range(4)]
        for i in range(4):
            y = (bias[i] + a[i][0] * xd[0] + a[i][1] * xd[1]
                 + a[i][2] * xd[2] + a[i][3] * xd[3])
            o_ref[sl, i, :] = y


def kernel(x, gamma, beta):
    batch, nfeat, dim = x.shape
    f32 = jnp.float32
    xt = jnp.transpose(x, (0, 2, 1))
    n_inner = batch // (_CORES * _BB)

    cparams = pltpu.CompilerParams(
        dimension_semantics=("parallel", "arbitrary"),
        vmem_limit_bytes=100 * 1024 * 1024)

    m1p, m2p = pl.pallas_call(
        _stats_kernel,
        grid=(_CORES, n_inner),
        in_specs=[pl.BlockSpec((_BB, 4, nfeat),
                               lambda c, j: (c * n_inner + j, 0, 0))],
        out_specs=[
            pl.BlockSpec((1, 8, nfeat), lambda c, j: (c, 0, 0)),
            pl.BlockSpec((1, 16, nfeat), lambda c, j: (c, 0, 0)),
        ],
        out_shape=[
            jax.ShapeDtypeStruct((_CORES, 8, nfeat), f32),
            jax.ShapeDtypeStruct((_CORES, 16, nfeat), f32),
        ],
        compiler_params=cparams,
    )(xt)
    m1 = m1p[0, :4] + m1p[1, :4]
    m2 = m2p[0] + m2p[1]

    p_in = jnp.concatenate(
        [m1, m2[:10], gamma.T, beta.T, jnp.zeros((4, nfeat), f32)], axis=0)

    q = pl.pallas_call(
        _make_solve_kernel(batch),
        out_shape=jax.ShapeDtypeStruct((24, nfeat), f32),
    )(p_in)

    cb = q[:20]

    out = pl.pallas_call(
        _apply_kernel,
        grid=(_CORES, n_inner),
        in_specs=[
            pl.BlockSpec((24, nfeat), lambda c, j: (0, 0)),
            pl.BlockSpec((_BB, 4, nfeat),
                         lambda c, j: (c * n_inner + j, 0, 0)),
        ],
        out_specs=pl.BlockSpec((_BB, 4, nfeat),
                               lambda c, j: (c * n_inner + j, 0, 0)),
        out_shape=jax.ShapeDtypeStruct((batch, 4, nfeat), f32),
        compiler_params=cparams,
    )(cb, xt)
    return jnp.transpose(out, (0, 2, 1))
